# baseline (speedup 1.0000x reference)
import functools

import jax
import jax.numpy as jnp
from jax.experimental import pallas as pl
from jax.experimental.pallas import tpu as pltpu


def _shared_kernel(x_ref, emb_tok_ref, wr_ref, w1s_ref, w2s_ref, gamma_ref,
                   out_ref, emb_ref, gate_ref):
    j = pl.program_id(0)

    @pl.when(j == 0)
    def _prologue():
        et = emb_tok_ref[...]
        var = jnp.mean(et * et, axis=-1, keepdims=True)
        emb_ref[...] = et * jax.lax.rsqrt(var + 1e-6) * gamma_ref[...]
        logits = jnp.dot(x_ref[...], wr_ref[...],
                         preferred_element_type=jnp.float32)
        m = jnp.max(logits, axis=-1, keepdims=True)
        p = jnp.exp(logits - m)
        gate_ref[...] = p / jnp.sum(p, axis=-1, keepdims=True)

    h = jax.nn.gelu(jnp.dot(x_ref[...], w1s_ref[...],
                            preferred_element_type=jnp.float32))
    contrib = jnp.dot(h, w2s_ref[...], preferred_element_type=jnp.float32)

    @pl.when(j == 0)
    def _():
        out_ref[...] = contrib

    @pl.when(j > 0)
    def _():
        out_ref[...] += contrib


def _routed_kernel(emb_ref, gate_ref, shared_ref, w1_ref, w2_ref, out_ref,
                   h_s, *, cpe, chunk):
    j = pl.program_id(0)
    e = j // cpe
    c2 = j % 2

    h = jax.nn.gelu(jnp.dot(emb_ref[...], w1_ref[0],
                            preferred_element_type=jnp.float32))
    n_e = gate_ref.shape[-1]
    mask = (jax.lax.broadcasted_iota(jnp.int32, (1, n_e), 1) == e)
    g = jnp.sum(gate_ref[...] * mask, axis=-1, keepdims=True)
    h_s[:, pl.ds(c2 * chunk, chunk)] = h * g

    @pl.when(c2 == 1)
    def _mm2():
        contrib = jnp.dot(h_s[...], w2_ref[0],
                          preferred_element_type=jnp.float32)

        @pl.when(j == 1)
        def _():
            out_ref[...] = shared_ref[...] + contrib

        @pl.when(j > 1)
        def _():
            out_ref[...] += contrib


def kernel(x, embed_tokens, W_r, W1s, W2s, W1, W2, gamma):
    B, T, D = x.shape
    E = W_r.shape[-1]
    DFF = W1s.shape[-1]
    N = B * T

    chunk = min(512, DFF)
    cpe = DFF // chunk

    x2 = x.reshape(N, D)
    emb2 = embed_tokens.reshape(N, D)
    gamma2 = gamma.reshape(1, D)

    shared_out, emb, gate = pl.pallas_call(
        _shared_kernel,
        grid=(cpe,),
        in_specs=[
            pl.BlockSpec((N, D), lambda j: (0, 0)),
            pl.BlockSpec((N, D), lambda j: (0, 0)),
            pl.BlockSpec((D, E), lambda j: (0, 0)),
            pl.BlockSpec((D, chunk), lambda j: (0, j)),
            pl.BlockSpec((chunk, D), lambda j: (j, 0)),
            pl.BlockSpec((1, D), lambda j: (0, 0)),
        ],
        out_specs=[
            pl.BlockSpec((N, D), lambda j: (0, 0)),
            pl.BlockSpec((N, D), lambda j: (0, 0)),
            pl.BlockSpec((N, E), lambda j: (0, 0)),
        ],
        out_shape=[
            jax.ShapeDtypeStruct((N, D), jnp.float32),
            jax.ShapeDtypeStruct((N, D), jnp.float32),
            jax.ShapeDtypeStruct((N, E), jnp.float32),
        ],
        compiler_params=pltpu.CompilerParams(
            dimension_semantics=("arbitrary",),
        ),
    )(x2, emb2, W_r, W1s, W2s, gamma2)

    out = pl.pallas_call(
        functools.partial(_routed_kernel, cpe=cpe, chunk=chunk),
        grid=(E * cpe,),
        in_specs=[
            pl.BlockSpec((N, D), lambda j: (0, 0)),
            pl.BlockSpec((N, E), lambda j: (0, 0)),
            pl.BlockSpec((N, D), lambda j: (0, 0)),
            pl.BlockSpec((1, D, chunk), lambda j: (j // cpe, 0, j % cpe)),
            pl.BlockSpec((1, 2 * chunk, D),
                         lambda j: (j // cpe, (j % cpe) // 2, 0)),
        ],
        out_specs=pl.BlockSpec((N, D), lambda j: (0, 0)),
        out_shape=jax.ShapeDtypeStruct((N, D), jnp.float32),
        scratch_shapes=[
            pltpu.VMEM((N, 2 * chunk), jnp.float32),
        ],
        compiler_params=pltpu.CompilerParams(
            dimension_semantics=("arbitrary",),
        ),
    )(emb, gate, shared_out, W1, W2)

    return out.reshape(B, T, D)


# speedup vs baseline: 1.1748x; 1.1748x over previous
import functools

import jax
import jax.numpy as jnp
from jax.experimental import pallas as pl
from jax.experimental.pallas import tpu as pltpu


def _mole_kernel(x_ref, emb_tok_ref, wr_ref, w1s_ref, w2s_ref, w1_ref, w2_ref,
                 gamma_ref, out_ref, emb_s, gate_s, *, n_shared, cpe):
    j = pl.program_id(1)

    @pl.when(j == 0)
    def _prologue():
        et = emb_tok_ref[...]
        var = jnp.mean(et * et, axis=-1, keepdims=True)
        emb_s[...] = et * jax.lax.rsqrt(var + 1e-6) * gamma_ref[...]
        logits = jnp.dot(x_ref[...], wr_ref[...],
                         preferred_element_type=jnp.float32)
        m = jnp.max(logits, axis=-1, keepdims=True)
        p = jnp.exp(logits - m)
        gate_s[...] = p / jnp.sum(p, axis=-1, keepdims=True)

    @pl.when(j < n_shared)
    def _shared_chunk():
        h = jax.nn.gelu(jnp.dot(x_ref[...], w1s_ref[...],
                                preferred_element_type=jnp.float32))
        contrib = jnp.dot(h, w2s_ref[...], preferred_element_type=jnp.float32)

        @pl.when(j == 0)
        def _():
            out_ref[...] = contrib

        @pl.when(j > 0)
        def _():
            out_ref[...] += contrib

    @pl.when(j >= n_shared)
    def _routed_chunk():
        e = (j - n_shared) // cpe
        h = jax.nn.gelu(jnp.dot(emb_s[...], w1_ref[0],
                                preferred_element_type=jnp.float32))
        n_e = gate_s.shape[-1]
        mask = (jax.lax.broadcasted_iota(jnp.int32, (1, n_e), 1) == e)
        g = jnp.sum(gate_s[...] * mask, axis=-1, keepdims=True)
        out_ref[...] += jnp.dot(h * g, w2_ref[0],
                                preferred_element_type=jnp.float32)


def kernel(x, embed_tokens, W_r, W1s, W2s, W1, W2, gamma):
    B, T, D = x.shape
    E = W_r.shape[-1]
    DFF = W1s.shape[-1]

    tokblk = min(2048, B * T)
    chunk = min(512, DFF)
    n_tok = (B * T) // tokblk
    cpe = DFF // chunk
    n_shared = cpe
    n_chunks = n_shared + E * cpe

    x2 = x.reshape(B * T, D)
    emb2 = embed_tokens.reshape(B * T, D)
    gamma2 = gamma.reshape(1, D)

    def jr(j):
        return jnp.maximum(j - n_shared, 0)

    out = pl.pallas_call(
        functools.partial(_mole_kernel, n_shared=n_shared, cpe=cpe),
        grid=(n_tok, n_chunks),
        in_specs=[
            pl.BlockSpec((tokblk, D), lambda t, j: (t, 0)),
            pl.BlockSpec((tokblk, D), lambda t, j: (t, 0)),
            pl.BlockSpec((D, E), lambda t, j: (0, 0)),
            pl.BlockSpec((D, chunk),
                         lambda t, j: (0, jnp.minimum(j, n_shared - 1))),
            pl.BlockSpec((chunk, D),
                         lambda t, j: (jnp.minimum(j, n_shared - 1), 0)),
            pl.BlockSpec((1, D, chunk),
                         lambda t, j: (jr(j) // cpe, 0, jr(j) % cpe)),
            pl.BlockSpec((1, chunk, D),
                         lambda t, j: (jr(j) // cpe, jr(j) % cpe, 0)),
            pl.BlockSpec((1, D), lambda t, j: (0, 0)),
        ],
        out_specs=pl.BlockSpec((tokblk, D), lambda t, j: (t, 0)),
        out_shape=jax.ShapeDtypeStruct((B * T, D), jnp.float32),
        scratch_shapes=[
            pltpu.VMEM((tokblk, D), jnp.float32),
            pltpu.VMEM((tokblk, E), jnp.float32),
        ],
        compiler_params=pltpu.CompilerParams(
            dimension_semantics=("arbitrary", "arbitrary"),
        ),
    )(x2, emb2, W_r, W1s, W2s, W1, W2, gamma2)

    return out.reshape(B, T, D)
